# baseline (speedup 1.0000x reference)
import numpy as np
import jax
import jax.numpy as jnp
from jax import lax
from jax.experimental import pallas as pl
from jax.experimental.pallas import tpu as pltpu
from jax.experimental.pallas import tpu_sc as plsc

_B, _C, _L = 16, 192, 4096
_S = 64
_ROWS = _B * _C
_NC, _NS = 2, 16
_NW = _NC * _NS
_RPW = _ROWS // _NW
_LANES = 16
_G = 4
_GL = _G * _L
_NG = _RPW // _G


def _morton_idx(l: int) -> np.ndarray:
    s = int(np.sqrt(l))
    ij = np.arange(l, dtype=np.int64)
    i = np.zeros(l, dtype=np.int64)
    j = np.zeros(l, dtype=np.int64)
    for t in range(int(l).bit_length() // 2 + 1):
        i += ((ij >> (2 * t + 1)) & 1) << t
        j += ((ij >> (2 * t)) & 1) << t
    return (i * s + j).astype(np.int64)


def _tables() -> np.ndarray:
    idx = _morton_idx(_L)
    lane = np.arange(_LANES)
    off = (lane & 1) * 1 + ((lane >> 1) & 1) * 4 + \
          ((lane >> 2) & 1) * 18 + ((lane >> 3) & 1) * 72
    free_positions = (5, 7, 8, 9, 10, 11)
    bases = []
    for hi in range(64):
        b = 0
        for t, p in enumerate(free_positions):
            b |= ((hi >> t) & 1) << p
        for q in (0, 16, 64, 80):
            bases.append(b | q)
    src = (np.array(sorted(bases))[:, None] ^ off[None, :]).reshape(-1)
    dst = idx[src]
    src_rows = np.concatenate([r * _L + src for r in range(_G)])
    dst_rows = np.concatenate([r * _L + dst for r in range(_G)])
    return np.concatenate([src_rows, dst_rows]).astype(np.int32)


_TABS_NP = _tables()


def _sc_body(x_hbm, tabs_hbm, out_hbm, srct_v, dstt_v, in_v, out_v,
             in_sem, out_sem):
    wid = lax.axis_index("s") * _NC + lax.axis_index("c")
    base = wid * _RPW * _L
    pltpu.sync_copy(tabs_hbm.at[pl.ds(0, _GL)], srct_v)
    pltpu.sync_copy(tabs_hbm.at[pl.ds(_GL, _GL)], dstt_v)

    def load(g, b):
        pltpu.async_copy(x_hbm.at[pl.ds(base + g * _GL, _GL)],
                         in_v.at[pl.ds(b * _GL, _GL)], in_sem.at[b])

    def wait_in(b):
        pltpu.make_async_copy(x_hbm.at[pl.ds(0, _GL)],
                              in_v.at[pl.ds(b * _GL, _GL)],
                              in_sem.at[b]).wait()

    def store(g, b):
        pltpu.async_copy(out_v.at[pl.ds(b * _GL, _GL)],
                         out_hbm.at[pl.ds(base + g * _GL, _GL)], out_sem.at[b])

    def wait_out(b):
        pltpu.make_async_copy(out_v.at[pl.ds(b * _GL, _GL)],
                              out_hbm.at[pl.ds(0, _GL)], out_sem.at[b]).wait()

    load(0, 0)

    @pl.loop(0, _NG, step=2)
    def _grp(g0):
        for b in range(2):
            g = g0 + b

            @pl.when(g + 1 < _NG)
            def _():
                load(g + 1, 1 - b)

            wait_in(b)

            @pl.when(g >= 2)
            def _():
                wait_out(b)

            in_b = in_v.at[pl.ds(b * _GL, _GL)]
            out_b = out_v.at[pl.ds(b * _GL, _GL)]

            @plsc.parallel_loop(0, _GL // _LANES, unroll=8)
            def _blk(k):
                p = k * _LANES
                iv_s = srct_v[pl.ds(p, _LANES)]
                iv_d = dstt_v[pl.ds(p, _LANES)]
                v = plsc.load_gather(in_b, [iv_s])
                plsc.store_scatter(out_b, [iv_d], v)

            store(g, b)

    wait_out(0)
    wait_out(1)


def kernel(x):
    xf = x.reshape(_ROWS * _L)
    tabs = jnp.asarray(_TABS_NP)
    mesh = plsc.VectorSubcoreMesh(core_axis_name="c", subcore_axis_name="s")
    out = pl.kernel(
        _sc_body,
        out_type=jax.ShapeDtypeStruct((_ROWS * _L,), jnp.float32),
        mesh=mesh,
        scratch_types=[
            pltpu.VMEM((_GL,), jnp.int32),
            pltpu.VMEM((_GL,), jnp.int32),
            pltpu.VMEM((2 * _GL,), jnp.float32),
            pltpu.VMEM((2 * _GL,), jnp.float32),
            pltpu.SemaphoreType.DMA((2,)),
            pltpu.SemaphoreType.DMA((2,)),
        ],
        compiler_params=pltpu.CompilerParams(needs_layout_passes=False),
    )(xf, tabs)
    return out.reshape(_B, _C, _S, _S)


# speedup vs baseline: 1.2292x; 1.2292x over previous
import numpy as np
import jax
import jax.numpy as jnp
from jax import lax
from jax.experimental import pallas as pl
from jax.experimental.pallas import tpu as pltpu
from jax.experimental.pallas import tpu_sc as plsc

_B, _C, _L = 16, 192, 4096
_S = 64
_ROWS = _B * _C
_NC, _NS = 2, 16
_NW = _NC * _NS
_RPW = _ROWS // _NW
_LANES = 16
_G = 4
_GL = _G * _L
_NG = _RPW // _G


def _morton_idx(l: int) -> np.ndarray:
    s = int(np.sqrt(l))
    ij = np.arange(l, dtype=np.int64)
    i = np.zeros(l, dtype=np.int64)
    j = np.zeros(l, dtype=np.int64)
    for t in range(int(l).bit_length() // 2 + 1):
        i += ((ij >> (2 * t + 1)) & 1) << t
        j += ((ij >> (2 * t)) & 1) << t
    return (i * s + j).astype(np.int32)


_IDX_NP = (_morton_idx(_L)[None, :] +
           (np.arange(2 * _G, dtype=np.int32) * _L)[:, None]).reshape(-1)


def _sc_body(x_hbm, idx_hbm, out_hbm, idx_v, in_v, out_v, in_sem, out_sem):
    wid = lax.axis_index("s") * _NC + lax.axis_index("c")
    row0 = wid * _RPW
    pltpu.sync_copy(idx_hbm, idx_v)

    def load(g, b):
        for r in range(_G):
            pltpu.async_copy(x_hbm.at[row0 + g * _G + r],
                             in_v.at[pl.ds((b * _G + r) * _L, _L)],
                             in_sem.at[b])

    def wait_in(b):
        for r in range(_G):
            pltpu.make_async_copy(x_hbm.at[0],
                                  in_v.at[pl.ds((b * _G + r) * _L, _L)],
                                  in_sem.at[b]).wait()

    def store(g, b):
        pltpu.async_copy(out_v.at[pl.ds(b * _GL, _GL)],
                         out_hbm.at[pl.ds((row0 + g * _G) * _L, _GL)],
                         out_sem.at[b])

    def wait_out(b):
        pltpu.make_async_copy(out_v.at[pl.ds(b * _GL, _GL)],
                              out_hbm.at[pl.ds(0, _GL)], out_sem.at[b]).wait()

    load(0, 0)

    @pl.loop(0, _NG, step=2)
    def _grp(g0):
        for b in range(2):
            g = g0 + b

            @pl.when(g + 1 < _NG)
            def _():
                load(g + 1, 1 - b)

            wait_in(b)

            @pl.when(g >= 2)
            def _():
                wait_out(b)

            o = b * _GL

            @plsc.parallel_loop(0, _GL // _LANES, unroll=8)
            def _blk(k):
                p = o + k * _LANES
                v = in_v[pl.ds(p, _LANES)]
                iv = idx_v[pl.ds(p, _LANES)]
                plsc.store_scatter(out_v, [iv], v)

            store(g, b)

    wait_out(0)
    wait_out(1)


def kernel(x):
    xf = x.reshape(_ROWS, _L)
    idx = jnp.asarray(_IDX_NP)
    mesh = plsc.VectorSubcoreMesh(core_axis_name="c", subcore_axis_name="s")
    out = pl.kernel(
        _sc_body,
        out_type=jax.ShapeDtypeStruct((_ROWS * _L,), jnp.float32),
        mesh=mesh,
        scratch_types=[
            pltpu.VMEM((2 * _GL,), jnp.int32),
            pltpu.VMEM((2 * _GL,), jnp.float32),
            pltpu.VMEM((2 * _GL,), jnp.float32),
            pltpu.SemaphoreType.DMA((2,)),
            pltpu.SemaphoreType.DMA((2,)),
        ],
        compiler_params=pltpu.CompilerParams(needs_layout_passes=False),
    )(xf, idx)
    return out.reshape(_B, _C, _S, _S)


# speedup vs baseline: 1.2524x; 1.0188x over previous
import numpy as np
import jax
import jax.numpy as jnp
from jax import lax
from jax.experimental import pallas as pl
from jax.experimental.pallas import tpu as pltpu
from jax.experimental.pallas import tpu_sc as plsc

_B, _C, _L = 16, 192, 4096
_S = 64
_ROWS = _B * _C
_NC, _NS = 2, 16
_NW = _NC * _NS
_RPW = _ROWS // _NW
_LANES = 16
_G = 6
_GL = _G * _L
_NG = _RPW // _G


def _morton_idx(l: int) -> np.ndarray:
    s = int(np.sqrt(l))
    ij = np.arange(l, dtype=np.int64)
    i = np.zeros(l, dtype=np.int64)
    j = np.zeros(l, dtype=np.int64)
    for t in range(int(l).bit_length() // 2 + 1):
        i += ((ij >> (2 * t + 1)) & 1) << t
        j += ((ij >> (2 * t)) & 1) << t
    return (i * s + j).astype(np.int32)


_IDX_NP = _morton_idx(_L)


def _sc_body(x_hbm, idx_hbm, out_hbm, idx_v, in_v, out_v, in_sem, out_sem):
    wid = lax.axis_index("s") * _NC + lax.axis_index("c")
    row0 = wid * _RPW
    pltpu.sync_copy(idx_hbm, idx_v)

    def load(g, b):
        for r in range(_G):
            pltpu.async_copy(x_hbm.at[row0 + g * _G + r],
                             in_v.at[pl.ds((b * _G + r) * _L, _L)],
                             in_sem.at[b])

    def wait_in(b):
        for r in range(_G):
            pltpu.make_async_copy(x_hbm.at[0],
                                  in_v.at[pl.ds((b * _G + r) * _L, _L)],
                                  in_sem.at[b]).wait()

    def store(g, b):
        pltpu.async_copy(out_v.at[pl.ds(b * _GL, _GL)],
                         out_hbm.at[pl.ds((row0 + g * _G) * _L, _GL)],
                         out_sem.at[b])

    def wait_out(b):
        pltpu.make_async_copy(out_v.at[pl.ds(b * _GL, _GL)],
                              out_hbm.at[pl.ds(0, _GL)], out_sem.at[b]).wait()

    load(0, 0)

    @pl.loop(0, _NG, step=2)
    def _grp(g0):
        for b in range(2):
            g = g0 + b

            @pl.when(g + 1 < _NG)
            def _():
                load(g + 1, 1 - b)

            wait_in(b)

            @pl.when(g >= 2)
            def _():
                wait_out(b)

            for r in range(_G):
                o = (b * _G + r) * _L

                @plsc.parallel_loop(0, _L // _LANES, unroll=8)
                def _blk(k):
                    p = k * _LANES
                    v = in_v[pl.ds(o + p, _LANES)]
                    iv = idx_v[pl.ds(p, _LANES)] + o
                    plsc.store_scatter(out_v, [iv], v)

            store(g, b)

    wait_out(0)
    wait_out(1)


def kernel(x):
    xf = x.reshape(_ROWS, _L)
    idx = jnp.asarray(_IDX_NP)
    mesh = plsc.VectorSubcoreMesh(core_axis_name="c", subcore_axis_name="s")
    out = pl.kernel(
        _sc_body,
        out_type=jax.ShapeDtypeStruct((_ROWS * _L,), jnp.float32),
        mesh=mesh,
        scratch_types=[
            pltpu.VMEM((_L,), jnp.int32),
            pltpu.VMEM((2 * _GL,), jnp.float32),
            pltpu.VMEM((2 * _GL,), jnp.float32),
            pltpu.SemaphoreType.DMA((2,)),
            pltpu.SemaphoreType.DMA((2,)),
        ],
        compiler_params=pltpu.CompilerParams(needs_layout_passes=False),
    )(xf, idx)
    return out.reshape(_B, _C, _S, _S)


# speedup vs baseline: 1.5239x; 1.2168x over previous
import numpy as np
import jax
import jax.numpy as jnp
from jax import lax
from jax.experimental import pallas as pl
from jax.experimental.pallas import tpu as pltpu
from jax.experimental.pallas import tpu_sc as plsc

_B, _C, _L = 16, 192, 4096
_S = 64
_ROWS = _B * _C
_NC, _NS = 2, 16
_NW = _NC * _NS
_RPW = _ROWS // _NW
_LANES = 16
_G = 6
_GL = _G * _L
_NG = _RPW // _G


def _morton_idx(l: int) -> np.ndarray:
    s = int(np.sqrt(l))
    ij = np.arange(l, dtype=np.int64)
    i = np.zeros(l, dtype=np.int64)
    j = np.zeros(l, dtype=np.int64)
    for t in range(int(l).bit_length() // 2 + 1):
        i += ((ij >> (2 * t + 1)) & 1) << t
        j += ((ij >> (2 * t)) & 1) << t
    return (i * s + j).astype(np.int32)


_IDX_NP = _morton_idx(_L)


def _sc_body(x_hbm, idx_hbm, out_hbm, idx_v, in_v, out_v, in_sem, out_sem):
    wid = lax.axis_index("s") * _NC + lax.axis_index("c")
    row0 = wid * _RPW
    pltpu.sync_copy(idx_hbm, idx_v)

    def load(g, b):
        for r in range(_G):
            pltpu.async_copy(x_hbm.at[row0 + g * _G + r],
                             in_v.at[pl.ds((b * _G + r) * _L, _L)],
                             in_sem.at[b])

    def wait_in(b):
        for r in range(_G):
            pltpu.make_async_copy(x_hbm.at[0],
                                  in_v.at[pl.ds((b * _G + r) * _L, _L)],
                                  in_sem.at[b]).wait()

    def store(g, b):
        for r in range(_G):
            pltpu.async_copy(out_v.at[pl.ds((b * _G + r) * _L, _L)],
                             out_hbm.at[row0 + g * _G + r], out_sem.at[b])

    def wait_out(b):
        for r in range(_G):
            pltpu.make_async_copy(out_v.at[pl.ds((b * _G + r) * _L, _L)],
                                  out_hbm.at[0], out_sem.at[b]).wait()

    load(0, 0)

    @pl.loop(0, _NG, step=2)
    def _grp(g0):
        for b in range(2):
            g = g0 + b

            @pl.when(g + 1 < _NG)
            def _():
                load(g + 1, 1 - b)

            wait_in(b)

            @pl.when(g >= 2)
            def _():
                wait_out(b)

            for r in range(_G):
                o = (b * _G + r) * _L

                @plsc.parallel_loop(0, _L // _LANES, unroll=8)
                def _blk(k):
                    p = k * _LANES
                    v = in_v[pl.ds(o + p, _LANES)]
                    iv = idx_v[pl.ds(p, _LANES)] + o
                    plsc.store_scatter(out_v, [iv], v)

            store(g, b)

    wait_out(0)
    wait_out(1)


def _sc_permute(xf, idx):
    mesh = plsc.VectorSubcoreMesh(core_axis_name="c", subcore_axis_name="s")
    return pl.kernel(
        _sc_body,
        out_type=jax.ShapeDtypeStruct((_ROWS, _L), jnp.float32),
        mesh=mesh,
        scratch_types=[
            pltpu.VMEM((_L,), jnp.int32),
            pltpu.VMEM((2 * _GL,), jnp.float32),
            pltpu.VMEM((2 * _GL,), jnp.float32),
            pltpu.SemaphoreType.DMA((2,)),
            pltpu.SemaphoreType.DMA((2,)),
        ],
        compiler_params=pltpu.CompilerParams(needs_layout_passes=False),
    )(xf, idx)


_CH = 512


def _tc_tr_body(z_ref, w_ref):
    w_ref[0] = jnp.swapaxes(z_ref[0], 0, 1)


def _tc_transpose(z3):
    return pl.pallas_call(
        _tc_tr_body,
        out_shape=jax.ShapeDtypeStruct((_B, _L, _C), jnp.float32),
        grid=(_B, _L // _CH),
        in_specs=[pl.BlockSpec((1, _C, _CH), lambda b, k: (b, 0, k))],
        out_specs=pl.BlockSpec((1, _CH, _C), lambda b, k: (b, k, 0)),
        compiler_params=pltpu.CompilerParams(
            dimension_semantics=("parallel", "parallel")),
    )(z3)


def kernel(x):
    xf = x.reshape(_ROWS, _L)
    idx = jnp.asarray(_IDX_NP)
    z = _sc_permute(xf, idx)
    w = _tc_transpose(z.reshape(_B, _C, _L))
    y = w.reshape(_B, _S, _S, _C)
    return y.transpose(0, 3, 1, 2)


# speedup vs baseline: 1.8466x; 1.2117x over previous
import numpy as np
import jax
import jax.numpy as jnp
from jax import lax
from jax.experimental import pallas as pl
from jax.experimental.pallas import tpu as pltpu
from jax.experimental.pallas import tpu_sc as plsc

_B, _C, _L = 16, 192, 4096
_S = 64
_ROWS = _B * _C
_NC, _NS = 2, 16
_NW = _NC * _NS
_RPW = _ROWS // _NW
_LANES = 16
_G = 6
_GL = _G * _L
_NG = _RPW // _G


def _morton_idx(l: int) -> np.ndarray:
    s = int(np.sqrt(l))
    ij = np.arange(l, dtype=np.int64)
    i = np.zeros(l, dtype=np.int64)
    j = np.zeros(l, dtype=np.int64)
    for t in range(int(l).bit_length() // 2 + 1):
        i += ((ij >> (2 * t + 1)) & 1) << t
        j += ((ij >> (2 * t)) & 1) << t
    return (i * s + j).astype(np.int32)


_IDX_NP = _morton_idx(_L)


def _sc_body(x_hbm, idx_hbm, out_hbm, idx_v, in_v, out_v, in_sem, out_sem):
    wid = lax.axis_index("s") * _NC + lax.axis_index("c")
    row0 = wid * _RPW
    pltpu.sync_copy(idx_hbm, idx_v)

    def load(g, b):
        for r in range(_G):
            pltpu.async_copy(x_hbm.at[row0 + g * _G + r],
                             in_v.at[pl.ds((b * _G + r) * _L, _L)],
                             in_sem.at[b])

    def wait_in(b):
        for r in range(_G):
            pltpu.make_async_copy(x_hbm.at[0],
                                  in_v.at[pl.ds((b * _G + r) * _L, _L)],
                                  in_sem.at[b]).wait()

    def store(g, b):
        for r in range(_G):
            pltpu.async_copy(out_v.at[pl.ds((b * _G + r) * _L, _L)],
                             out_hbm.at[row0 + g * _G + r], out_sem.at[b])

    def wait_out(b):
        for r in range(_G):
            pltpu.make_async_copy(out_v.at[pl.ds((b * _G + r) * _L, _L)],
                                  out_hbm.at[0], out_sem.at[b]).wait()

    load(0, 0)

    @pl.loop(0, _NG, step=2)
    def _grp(g0):
        for b in range(2):
            g = g0 + b

            @pl.when(g + 1 < _NG)
            def _():
                load(g + 1, 1 - b)

            wait_in(b)

            @pl.when(g >= 2)
            def _():
                wait_out(b)

            for r in range(_G):
                o = (b * _G + r) * _L

                @plsc.parallel_loop(0, _L // _LANES, unroll=8)
                def _blk(k):
                    p = k * _LANES
                    v = in_v[pl.ds(o + p, _LANES)]
                    iv = idx_v[pl.ds(p, _LANES)] + o
                    plsc.store_scatter(out_v, [iv], v)

            store(g, b)

    wait_out(0)
    wait_out(1)


def _sc_permute(xf, idx):
    mesh = plsc.VectorSubcoreMesh(core_axis_name="c", subcore_axis_name="s")
    return pl.kernel(
        _sc_body,
        out_type=jax.ShapeDtypeStruct((_ROWS, _L), jnp.float32),
        mesh=mesh,
        scratch_types=[
            pltpu.VMEM((_L,), jnp.int32),
            pltpu.VMEM((2 * _GL,), jnp.float32),
            pltpu.VMEM((2 * _GL,), jnp.float32),
            pltpu.SemaphoreType.DMA((2,)),
            pltpu.SemaphoreType.DMA((2,)),
        ],
        compiler_params=pltpu.CompilerParams(needs_layout_passes=False),
    )(xf, idx)


_CH = 1024


def _tc_tr_body(z_ref, w_ref):
    zb = z_ref[0]
    r = lax.broadcasted_iota(jnp.int32, (_C, _C), 0)
    c = lax.broadcasted_iota(jnp.int32, (_C, _C), 1)
    eye = (r == c).astype(jnp.float32)
    w_ref[0] = lax.dot_general(zb, eye, (((0,), (0,)), ((), ())),
                               preferred_element_type=jnp.float32)


def _tc_transpose(z3):
    return pl.pallas_call(
        _tc_tr_body,
        out_shape=jax.ShapeDtypeStruct((_B, _L, _C), jnp.float32),
        grid=(_B, _L // _CH),
        in_specs=[pl.BlockSpec((1, _C, _CH), lambda b, k: (b, 0, k))],
        out_specs=pl.BlockSpec((1, _CH, _C), lambda b, k: (b, k, 0)),
        compiler_params=pltpu.CompilerParams(
            dimension_semantics=("parallel", "parallel")),
    )(z3)


def kernel(x):
    xf = x.reshape(_ROWS, _L)
    idx = jnp.asarray(_IDX_NP)
    z = _sc_permute(xf, idx)
    w = _tc_transpose(z.reshape(_B, _C, _L))
    y = w.reshape(_B, _S, _S, _C)
    return y.transpose(0, 3, 1, 2)
